# baseline (speedup 1.0000x reference)
import functools
import math

import jax
import jax.numpy as jnp
import numpy as np
from jax import lax
from jax.experimental import pallas as pl
from jax.experimental.pallas import tpu as pltpu
from jax.experimental.pallas import tpu_sc as plsc

_NI = 128
_ND = 128
_NH = 4
_KN = 32
_SDK = math.sqrt(float(_ND))
_BN = 80
_SB = 16
_CH = 80
_CHUNKS = (2000, 2000, 2000, 2000, 2000)


def _elu(t):
    return jnp.where(t > 0, t, jnp.exp(jnp.minimum(t, 0.0)) - 1.0)


def _np_consts():
    pc = np.zeros((3, 16, 16), np.float32)
    pd = np.zeros((3, 16, 16), np.float32)
    for b in range(3):
        for j in range(12):
            pc[b, 3 + 3 * (j % 3) + b, j] = 1.0
            pd[b, 3 * (j // 3) + b, j] = 1.0
    bs = np.zeros((16, 128), np.float32)
    bs[0, :] = 1.0
    m012 = np.zeros((1, 16), np.float32)
    m012[0, :3] = 1.0
    return pc, pd, bs, m012


_PC, _PD, _BS128, _M012 = _np_consts()


def _sc_gather(ftab, gtab, idx):
    B = idx.shape[0]
    info = plsc.get_sparse_core_info()
    NC, NS = info.num_cores, info.num_subcores
    NW = NC * NS
    bpw = B // NW
    nch = bpw // _CH
    mesh = plsc.VectorSubcoreMesh(core_axis_name="c", subcore_axis_name="s")

    @functools.partial(
        pl.kernel,
        mesh=mesh,
        out_type=[
            jax.ShapeDtypeStruct((B, 128), jnp.float32),
            jax.ShapeDtypeStruct((B, 16), jnp.float32),
        ],
        scratch_types=[
            pltpu.VMEM((bpw,), jnp.int32),
            pltpu.VMEM((_CH, 128), jnp.float32),
            pltpu.VMEM((_CH, 128), jnp.float32),
            pltpu.VMEM((_CH, 16), jnp.float32),
            pltpu.SemaphoreType.DMA,
            pltpu.SemaphoreType.DMA,
        ],
    )
    def k(ftab_hbm, gtab_hbm, idx_hbm, outf, outg, idx_v, frows, grows, gcomp, sf, sg):
        wid = lax.axis_index("s") * NC + lax.axis_index("c")
        base = wid * bpw
        pltpu.sync_copy(idx_hbm.at[pl.ds(base, bpw)], idx_v)

        def body(c, carry):
            off = c * _CH
            cpf = pltpu.async_copy(ftab_hbm.at[idx_v.at[pl.ds(off, _CH)]], frows, sf)
            cpg = pltpu.async_copy(gtab_hbm.at[idx_v.at[pl.ds(off, _CH)]], grows, sg)
            cpf.wait()
            cpg.wait()

            def compact(i, cc):
                gcomp[i, :] = grows[i, 0:16]
                return cc

            lax.fori_loop(0, _CH, compact, 0)
            pltpu.sync_copy(frows, outf.at[pl.ds(base + off, _CH)])
            pltpu.sync_copy(gcomp, outg.at[pl.ds(base + off, _CH)])
            return carry

        lax.fori_loop(0, nch, body, 0)

    return k(ftab, gtab, idx)


def _tc_body(f_ref, gf_ref, gg_ref, cg_ref, p32_ref, tk_ref,
             pc0, pc1, pc2, pd0, pd1, pd2, bs_ref, m012_ref,
             qw1, qb1, qw2, qb2, gw1, gb1, gw2, gb2,
             kw1, kb1, kw2, kb2, vw1, vb1, vw2, vb2,
             dw1, db1, dw2, db2, lg, lb,
             o_ref, *, sub):
    f = f_ref[...]
    dot = functools.partial(jnp.dot, preferred_element_type=jnp.float32)
    b16 = lambda a: a.astype(jnp.bfloat16)

    def bdot(a, w):
        return dot(b16(a), b16(w))

    t = _elu(bdot(f, qw1[...]) + qb1[...])
    qq = bdot(t, qw2[...]) + qb2[...]

    cg = cg_ref[...]
    e = dot(p32_ref[...], cg)
    gg = gg_ref[...]
    gp = gg - e * m012_ref[...]
    a0 = dot(e, pc0[...])
    a1 = dot(e, pc1[...])
    a2 = dot(e, pc2[...])
    d0 = dot(gp, pd0[...])
    d1 = dot(gp, pd1[...])
    d2 = dot(gp, pd2[...])
    rl = a0 * d0 + a1 * d1 + a2 * d2
    geo_pre = dot(rl, gw1[...]) + gb1[...]

    sq128 = dot(d0 * d0 + d1 * d1 + d2 * d2, bs_ref[...])
    dis128 = jnp.exp(-0.5 * sq128)

    gfl = bdot(_elu(geo_pre), gw2[...]) + gb2[...]
    gfl = gfl * dis128 * gf_ref[...]

    inv = 1.0 / _SDK
    rows, cols = _NH * _SB, _SB * _KN
    ri = lax.broadcasted_iota(jnp.int32, (rows, cols), 0)
    ci = lax.broadcasted_iota(jnp.int32, (rows, cols), 1)
    smask = (ri % _SB) == (ci // _KN)

    k_all = bdot(_elu(bdot(gfl, kw1[...]) + kb1[...]), kw2[...]) + kb2[...]
    v_all = bdot(_elu(bdot(gfl, vw1[...]) + vb1[...]), vw2[...]) + vb2[...]

    def att(s):
        q = lax.slice(qq, (s * _SB, 0), (s * _SB + _SB, 512))
        qs = jnp.concatenate([q[:, 128 * h:128 * (h + 1)] for h in range(_NH)], axis=0)
        kk = lax.slice(k_all, (s * cols, 0), (s * cols + cols, 128))
        vv = lax.slice(v_all, (s * cols, 0), (s * cols + cols, 128))
        sc = lax.dot_general(b16(qs), b16(kk), (((1,), (1,)), ((), ())),
                             preferred_element_type=jnp.float32) * inv
        tz = tk_ref[0, s, :][None, :]
        valid = smask & (tz != 0)
        m = jnp.max(sc, axis=1, keepdims=True)
        e = jnp.exp(sc - m) * valid.astype(jnp.float32)
        den = jnp.sum(e, axis=1, keepdims=True)
        o = lax.dot_general(b16(e), b16(vv), (((1,), (0,)), ((), ())),
                            preferred_element_type=jnp.float32) / den
        return jnp.concatenate(
            [o[_SB * h:_SB * (h + 1), :] for h in range(_NH)], axis=1)

    at = jnp.concatenate([att(s) for s in range(sub)], axis=0)

    o = bdot(_elu(bdot(at, dw1[...]) + db1[...]), dw2[...]) + db2[...] + f
    mu = jnp.mean(o, axis=1, keepdims=True)
    d = o - mu
    var = jnp.mean(d * d, axis=1, keepdims=True)
    o_ref[...] = d * lax.rsqrt(var + 1e-5) * lg[...] + lb[...]


def _tc_kwargs(n, k):
    nb = n // _BN
    sub = _BN // _SB
    cm = lambda i: (0, 0)
    in_specs = [
        pl.BlockSpec((_BN, 128), lambda i: (i, 0)),
        pl.BlockSpec((_BN * k, 128), lambda i: (i, 0)),
        pl.BlockSpec((_BN * k, 16), lambda i: (i, 0)),
        pl.BlockSpec((_BN, 16), lambda i: (i, 0)),
        pl.BlockSpec((_BN * k, _BN), cm),
        pl.BlockSpec((1, sub, _SB * k), lambda i: (i, 0, 0)),
        pl.BlockSpec((16, 16), cm), pl.BlockSpec((16, 16), cm),
        pl.BlockSpec((16, 16), cm), pl.BlockSpec((16, 16), cm),
        pl.BlockSpec((16, 16), cm), pl.BlockSpec((16, 16), cm),
        pl.BlockSpec((16, 128), cm), pl.BlockSpec((1, 16), cm),
        pl.BlockSpec((128, 128), cm), pl.BlockSpec((1, 128), cm),
        pl.BlockSpec((128, 512), cm), pl.BlockSpec((1, 512), cm),
        pl.BlockSpec((16, 128), cm), pl.BlockSpec((1, 128), cm),
        pl.BlockSpec((128, 128), cm), pl.BlockSpec((1, 128), cm),
        pl.BlockSpec((128, 128), cm), pl.BlockSpec((1, 128), cm),
        pl.BlockSpec((128, 128), cm), pl.BlockSpec((1, 128), cm),
        pl.BlockSpec((128, 128), cm), pl.BlockSpec((1, 128), cm),
        pl.BlockSpec((128, 128), cm), pl.BlockSpec((1, 128), cm),
        pl.BlockSpec((512, 128), cm), pl.BlockSpec((1, 128), cm),
        pl.BlockSpec((128, 128), cm), pl.BlockSpec((1, 128), cm),
        pl.BlockSpec((1, 128), cm), pl.BlockSpec((1, 128), cm),
    ]
    return dict(
        grid=(nb,),
        in_specs=in_specs,
        out_specs=pl.BlockSpec((_BN, 128), lambda i: (i, 0)),
        out_shape=jax.ShapeDtypeStruct((n, 128), jnp.float32),
    )


def kernel(features, x, nuv, topk, qf_w1, qf_b1, qf_w2, qf_b2,
           ge_w1, ge_b1, ge_w2, ge_b2, kf_w1, kf_b1, kf_w2, kf_b2,
           vf_w1, vf_b1, vf_w2, vf_b2, df_w1, df_b1, df_w2, df_b2,
           ln_g, ln_b):
    n = features.shape[0]
    k = topk.shape[1]
    f32 = jnp.float32

    geo12 = jnp.concatenate([x.astype(f32), nuv.reshape(n, 9).astype(f32)], axis=1)
    gtab = jnp.concatenate([geo12, jnp.zeros((n, 116), f32)], axis=1)
    ctab = jnp.concatenate([geo12, jnp.zeros((n, 4), f32)], axis=1)
    idx = topk.reshape(-1).astype(jnp.int32)

    sub = _BN // _SB
    tki = topk.astype(jnp.int32).reshape(n // _BN, sub, _SB * k)
    gw1p = jnp.concatenate([ge_w1, jnp.zeros((4, 128), f32)], axis=0)
    p32 = jnp.asarray(
        (np.arange(_BN * k)[:, None] // k == np.arange(_BN)[None, :]).astype(np.float32))
    row = lambda b: b.reshape(1, -1)

    body = functools.partial(_tc_body, sub=sub)
    outs = []
    n0 = 0
    for nc in _CHUNKS:
        gf, gg = _sc_gather(features, gtab, idx[n0 * k:(n0 + nc) * k])
        out_c = pl.pallas_call(body, **_tc_kwargs(nc, k))(
            lax.slice_in_dim(features, n0, n0 + nc),
            gf, gg,
            lax.slice_in_dim(ctab, n0, n0 + nc),
            p32, lax.slice_in_dim(tki, n0 // _BN, (n0 + nc) // _BN),
            jnp.asarray(_PC[0]), jnp.asarray(_PC[1]), jnp.asarray(_PC[2]),
            jnp.asarray(_PD[0]), jnp.asarray(_PD[1]), jnp.asarray(_PD[2]),
            jnp.asarray(_BS128), jnp.asarray(_M012),
            qf_w1, row(qf_b1), qf_w2, row(qf_b2),
            gw1p, row(ge_b1), ge_w2, row(ge_b2),
            kf_w1, row(kf_b1), kf_w2, row(kf_b2),
            vf_w1, row(vf_b1), vf_w2, row(vf_b2),
            df_w1, row(df_b1), df_w2, row(df_b2),
            row(ln_g), row(ln_b))
        outs.append(out_c)
        n0 += nc
    return jnp.concatenate(outs, axis=0)


# speedup vs baseline: 1.1042x; 1.1042x over previous
import functools
import math

import jax
import jax.numpy as jnp
import numpy as np
from jax import lax
from jax.experimental import pallas as pl
from jax.experimental.pallas import tpu as pltpu
from jax.experimental.pallas import tpu_sc as plsc

_NI = 128
_ND = 128
_NH = 4
_KN = 32
_SDK = math.sqrt(float(_ND))
_BN = 80
_SB = 16
_CH = 80
_CHUNKS = (2000, 2000, 2000, 2000, 2000)


def _elu(t):
    return jnp.where(t > 0, t, jnp.exp(jnp.minimum(t, 0.0)) - 1.0)


def _np_consts():
    pcc = np.zeros((16, 48), np.float32)
    pdc = np.zeros((16, 48), np.float32)
    s48 = np.zeros((48, 16), np.float32)
    for b in range(3):
        for j in range(12):
            pcc[3 + 3 * (j % 3) + b, 16 * b + j] = 1.0
            pdc[3 * (j // 3) + b, 16 * b + j] = 1.0
            s48[16 * b + j, j] = 1.0
    bs48 = np.zeros((48, 128), np.float32)
    bs48[0, :] = bs48[16, :] = bs48[32, :] = 1.0
    m012 = np.zeros((1, 16), np.float32)
    m012[0, :3] = 1.0
    return pcc, pdc, s48, bs48, m012


_PCC, _PDC, _S48, _BS48, _M012 = _np_consts()


def _sc_gather(ftab, gtab, idx):
    B = idx.shape[0]
    info = plsc.get_sparse_core_info()
    NC, NS = info.num_cores, info.num_subcores
    NW = NC * NS
    bpw = B // NW
    nch = bpw // _CH
    mesh = plsc.VectorSubcoreMesh(core_axis_name="c", subcore_axis_name="s")

    @functools.partial(
        pl.kernel,
        mesh=mesh,
        out_type=[
            jax.ShapeDtypeStruct((B, 128), jnp.float32),
            jax.ShapeDtypeStruct((B, 16), jnp.float32),
        ],
        scratch_types=[
            pltpu.VMEM((bpw,), jnp.int32),
            pltpu.VMEM((_CH, 128), jnp.float32),
            pltpu.VMEM((_CH, 128), jnp.float32),
            pltpu.VMEM((_CH, 16), jnp.float32),
            pltpu.SemaphoreType.DMA,
            pltpu.SemaphoreType.DMA,
        ],
    )
    def k(ftab_hbm, gtab_hbm, idx_hbm, outf, outg, idx_v, frows, grows, gcomp, sf, sg):
        wid = lax.axis_index("s") * NC + lax.axis_index("c")
        base = wid * bpw
        pltpu.sync_copy(idx_hbm.at[pl.ds(base, bpw)], idx_v)

        def body(c, carry):
            off = c * _CH
            cpf = pltpu.async_copy(ftab_hbm.at[idx_v.at[pl.ds(off, _CH)]], frows, sf)
            cpg = pltpu.async_copy(gtab_hbm.at[idx_v.at[pl.ds(off, _CH)]], grows, sg)
            cpf.wait()
            cpg.wait()

            def compact(i, cc):
                gcomp[i, :] = grows[i, 0:16]
                return cc

            lax.fori_loop(0, _CH, compact, 0)
            pltpu.sync_copy(frows, outf.at[pl.ds(base + off, _CH)])
            pltpu.sync_copy(gcomp, outg.at[pl.ds(base + off, _CH)])
            return carry

        lax.fori_loop(0, nch, body, 0)

    return k(ftab, gtab, idx)


def _tc_body(f_ref, gf_ref, gg_ref, cg_ref, p32_ref, tk_ref,
             pcc, pdc, s48, bs48, m012_ref,
             qw1, qb1, qw2, qb2, gw1, gb1, gw2, gb2,
             kw1, kb1, kw2, kb2, vw1, vb1, vw2, vb2,
             dw1, db1, dw2, db2, lg, lb,
             o_ref, *, sub):
    f = f_ref[...]
    dot = functools.partial(jnp.dot, preferred_element_type=jnp.float32)

    t = _elu(dot(f, qw1[...]) + qb1[...])
    qq = dot(t, qw2[...]) + qb2[...]

    cg = cg_ref[...]
    e = dot(p32_ref[...], cg)
    gg = gg_ref[...]
    gp = gg - e * m012_ref[...]
    ac = dot(e, pcc[...])
    dc = dot(gp, pdc[...])
    rl = dot(ac * dc, s48[...])
    geo_pre = dot(rl, gw1[...]) + gb1[...]

    sq128 = dot(dc * dc, bs48[...])
    dis128 = jnp.exp(-0.5 * sq128)

    gfl = dot(_elu(geo_pre), gw2[...]) + gb2[...]
    gfl = gfl * dis128 * gf_ref[...]

    inv = 1.0 / _SDK
    rows, cols = _NH * _SB, _SB * _KN
    ri = lax.broadcasted_iota(jnp.int32, (rows, cols), 0)
    ci = lax.broadcasted_iota(jnp.int32, (rows, cols), 1)
    smask = (ri % _SB) == (ci // _KN)

    k_all = dot(_elu(dot(gfl, kw1[...]) + kb1[...]), kw2[...]) + kb2[...]
    v_all = dot(_elu(dot(gfl, vw1[...]) + vb1[...]), vw2[...]) + vb2[...]

    def att(s):
        q = lax.slice(qq, (s * _SB, 0), (s * _SB + _SB, 512))
        qs = jnp.concatenate([q[:, 128 * h:128 * (h + 1)] for h in range(_NH)], axis=0)
        kk = lax.slice(k_all, (s * cols, 0), (s * cols + cols, 128))
        vv = lax.slice(v_all, (s * cols, 0), (s * cols + cols, 128))
        sc = lax.dot_general(qs, kk, (((1,), (1,)), ((), ())),
                             preferred_element_type=jnp.float32) * inv
        tz = tk_ref[0, s, :][None, :]
        valid = smask & (tz != 0)
        m = jnp.max(sc, axis=1, keepdims=True)
        e = jnp.exp(sc - m) * valid.astype(jnp.float32)
        den = jnp.sum(e, axis=1, keepdims=True)
        o = lax.dot_general(e, vv, (((1,), (0,)), ((), ())),
                            preferred_element_type=jnp.float32) / den
        return jnp.concatenate(
            [o[_SB * h:_SB * (h + 1), :] for h in range(_NH)], axis=1)

    at = jnp.concatenate([att(s) for s in range(sub)], axis=0)

    o = dot(_elu(dot(at, dw1[...]) + db1[...]), dw2[...]) + db2[...] + f
    mu = jnp.mean(o, axis=1, keepdims=True)
    d = o - mu
    var = jnp.mean(d * d, axis=1, keepdims=True)
    o_ref[...] = d * lax.rsqrt(var + 1e-5) * lg[...] + lb[...]


def _tc_kwargs(n, k):
    nb = n // _BN
    sub = _BN // _SB
    cm = lambda i: (0, 0)
    in_specs = [
        pl.BlockSpec((_BN, 128), lambda i: (i, 0)),
        pl.BlockSpec((_BN * k, 128), lambda i: (i, 0)),
        pl.BlockSpec((_BN * k, 16), lambda i: (i, 0)),
        pl.BlockSpec((_BN, 16), lambda i: (i, 0)),
        pl.BlockSpec((_BN * k, _BN), cm),
        pl.BlockSpec((1, sub, _SB * k), lambda i: (i, 0, 0)),
        pl.BlockSpec((16, 48), cm), pl.BlockSpec((16, 48), cm),
        pl.BlockSpec((48, 16), cm), pl.BlockSpec((48, 128), cm),
        pl.BlockSpec((1, 16), cm),
        pl.BlockSpec((128, 128), cm), pl.BlockSpec((1, 128), cm),
        pl.BlockSpec((128, 512), cm), pl.BlockSpec((1, 512), cm),
        pl.BlockSpec((16, 128), cm), pl.BlockSpec((1, 128), cm),
        pl.BlockSpec((128, 128), cm), pl.BlockSpec((1, 128), cm),
        pl.BlockSpec((128, 128), cm), pl.BlockSpec((1, 128), cm),
        pl.BlockSpec((128, 128), cm), pl.BlockSpec((1, 128), cm),
        pl.BlockSpec((128, 128), cm), pl.BlockSpec((1, 128), cm),
        pl.BlockSpec((128, 128), cm), pl.BlockSpec((1, 128), cm),
        pl.BlockSpec((512, 128), cm), pl.BlockSpec((1, 128), cm),
        pl.BlockSpec((128, 128), cm), pl.BlockSpec((1, 128), cm),
        pl.BlockSpec((1, 128), cm), pl.BlockSpec((1, 128), cm),
    ]
    return dict(
        grid=(nb,),
        in_specs=in_specs,
        out_specs=pl.BlockSpec((_BN, 128), lambda i: (i, 0)),
        out_shape=jax.ShapeDtypeStruct((n, 128), jnp.float32),
    )


def kernel(features, x, nuv, topk, qf_w1, qf_b1, qf_w2, qf_b2,
           ge_w1, ge_b1, ge_w2, ge_b2, kf_w1, kf_b1, kf_w2, kf_b2,
           vf_w1, vf_b1, vf_w2, vf_b2, df_w1, df_b1, df_w2, df_b2,
           ln_g, ln_b):
    n = features.shape[0]
    k = topk.shape[1]
    f32 = jnp.float32

    geo12 = jnp.concatenate([x.astype(f32), nuv.reshape(n, 9).astype(f32)], axis=1)
    gtab = jnp.concatenate([geo12, jnp.zeros((n, 116), f32)], axis=1)
    ctab = jnp.concatenate([geo12, jnp.zeros((n, 4), f32)], axis=1)
    idx = topk.reshape(-1).astype(jnp.int32)

    sub = _BN // _SB
    tki = topk.astype(jnp.int32).reshape(n // _BN, sub, _SB * k)
    gw1p = jnp.concatenate([ge_w1, jnp.zeros((4, 128), f32)], axis=0)
    p32 = jnp.asarray(
        (np.arange(_BN * k)[:, None] // k == np.arange(_BN)[None, :]).astype(np.float32))
    row = lambda b: b.reshape(1, -1)

    body = functools.partial(_tc_body, sub=sub)
    outs = []
    n0 = 0
    for nc in _CHUNKS:
        gf, gg = _sc_gather(features, gtab, idx[n0 * k:(n0 + nc) * k])
        out_c = pl.pallas_call(body, **_tc_kwargs(nc, k))(
            lax.slice_in_dim(features, n0, n0 + nc),
            gf, gg,
            lax.slice_in_dim(ctab, n0, n0 + nc),
            p32, lax.slice_in_dim(tki, n0 // _BN, (n0 + nc) // _BN),
            jnp.asarray(_PCC), jnp.asarray(_PDC),
            jnp.asarray(_S48), jnp.asarray(_BS48), jnp.asarray(_M012),
            qf_w1, row(qf_b1), qf_w2, row(qf_b2),
            gw1p, row(ge_b1), ge_w2, row(ge_b2),
            kf_w1, row(kf_b1), kf_w2, row(kf_b2),
            vf_w1, row(vf_b1), vf_w2, row(vf_b2),
            df_w1, row(df_b1), df_w2, row(df_b2),
            row(ln_g), row(ln_b))
        outs.append(out_c)
        n0 += nc
    return jnp.concatenate(outs, axis=0)


# speedup vs baseline: 1.1075x; 1.0029x over previous
import functools
import math

import jax
import jax.numpy as jnp
import numpy as np
from jax import lax
from jax.experimental import pallas as pl
from jax.experimental.pallas import tpu as pltpu
from jax.experimental.pallas import tpu_sc as plsc

_NI = 128
_ND = 128
_NH = 4
_KN = 32
_SDK = math.sqrt(float(_ND))
_BN = 80
_SB = 16
_CH = 80
_CHUNKS = (2000, 2000, 2000, 2000, 2000)


def _elu(t):
    return jnp.where(t > 0, t, jnp.exp(jnp.minimum(t, 0.0)) - 1.0)


def _np_consts():
    pcc = np.zeros((16, 48), np.float32)
    pdc = np.zeros((16, 48), np.float32)
    s48 = np.zeros((48, 16), np.float32)
    for b in range(3):
        for j in range(12):
            pcc[3 + 3 * (j % 3) + b, 16 * b + j] = 1.0
            pdc[3 * (j // 3) + b, 16 * b + j] = 1.0
            s48[16 * b + j, j] = 1.0
    bs48 = np.zeros((48, 128), np.float32)
    bs48[0, :] = bs48[16, :] = bs48[32, :] = -0.5
    m012 = np.zeros((1, 16), np.float32)
    m012[0, :3] = 1.0
    return pcc, pdc, s48, bs48, m012


_PCC, _PDC, _S48, _BS48, _M012 = _np_consts()


def _sc_gather(ftab, gtab, idx):
    B = idx.shape[0]
    info = plsc.get_sparse_core_info()
    NC, NS = info.num_cores, info.num_subcores
    NW = NC * NS
    bpw = B // NW
    nch = bpw // _CH
    mesh = plsc.VectorSubcoreMesh(core_axis_name="c", subcore_axis_name="s")

    @functools.partial(
        pl.kernel,
        mesh=mesh,
        out_type=[
            jax.ShapeDtypeStruct((B, 128), jnp.float32),
            jax.ShapeDtypeStruct((B, 16), jnp.float32),
        ],
        scratch_types=[
            pltpu.VMEM((bpw,), jnp.int32),
            pltpu.VMEM((_CH, 128), jnp.float32),
            pltpu.VMEM((_CH, 128), jnp.float32),
            pltpu.VMEM((_CH, 16), jnp.float32),
            pltpu.SemaphoreType.DMA,
            pltpu.SemaphoreType.DMA,
        ],
    )
    def k(ftab_hbm, gtab_hbm, idx_hbm, outf, outg, idx_v, frows, grows, gcomp, sf, sg):
        wid = lax.axis_index("s") * NC + lax.axis_index("c")
        base = wid * bpw
        pltpu.sync_copy(idx_hbm.at[pl.ds(base, bpw)], idx_v)

        def body(c, carry):
            off = c * _CH
            cpf = pltpu.async_copy(ftab_hbm.at[idx_v.at[pl.ds(off, _CH)]], frows, sf)
            cpg = pltpu.async_copy(gtab_hbm.at[idx_v.at[pl.ds(off, _CH)]], grows, sg)
            cpf.wait()
            cpg.wait()

            def compact(i, cc):
                gcomp[i, :] = grows[i, 0:16]
                return cc

            lax.fori_loop(0, _CH, compact, 0)
            pltpu.sync_copy(frows, outf.at[pl.ds(base + off, _CH)])
            pltpu.sync_copy(gcomp, outg.at[pl.ds(base + off, _CH)])
            return carry

        lax.fori_loop(0, nch, body, 0)

    return k(ftab, gtab, idx)


def _tc_body(f_ref, gf_ref, gg_ref, cg_ref, p32_ref, tk_ref,
             pcc, pdc, s48, bs48, m012_ref,
             qw1, qb1, qw2, qb2, gw1, gb1, gw2, gb2,
             kw1, kb1, kw2, kb2, vw1, vb1, vw2, vb2,
             dw1, db1, dw2, db2, lg, lb,
             o_ref, *, sub):
    f = f_ref[...]
    dot = functools.partial(jnp.dot, preferred_element_type=jnp.float32)

    t = _elu(dot(f, qw1[...]) + qb1[...])
    qq = (dot(t, qw2[...]) + qb2[...]) * (1.0 / _SDK)

    cg = cg_ref[...]
    e = dot(p32_ref[...], cg)
    gg = gg_ref[...]
    gp = gg - e * m012_ref[...]
    ac = dot(e, pcc[...])
    dc = dot(gp, pdc[...])
    rl = dot(ac * dc, s48[...])
    geo_pre = dot(rl, gw1[...]) + gb1[...]

    dis128 = jnp.exp(dot(dc * dc, bs48[...]))

    gfl = dot(_elu(geo_pre), gw2[...]) + gb2[...]
    gfl = gfl * dis128 * gf_ref[...]

    rows, cols = _NH * _SB, _SB * _KN
    ri = lax.broadcasted_iota(jnp.int32, (rows, cols), 0)
    ci = lax.broadcasted_iota(jnp.int32, (rows, cols), 1)
    smask = (ri % _SB) == (ci // _KN)

    k_all = dot(_elu(dot(gfl, kw1[...]) + kb1[...]), kw2[...]) + kb2[...]
    v_all = dot(_elu(dot(gfl, vw1[...]) + vb1[...]), vw2[...]) + vb2[...]

    def att(s):
        q = lax.slice(qq, (s * _SB, 0), (s * _SB + _SB, 512))
        qs = jnp.concatenate([q[:, 128 * h:128 * (h + 1)] for h in range(_NH)], axis=0)
        kk = lax.slice(k_all, (s * cols, 0), (s * cols + cols, 128))
        vv = lax.slice(v_all, (s * cols, 0), (s * cols + cols, 128))
        sc = lax.dot_general(qs, kk, (((1,), (1,)), ((), ())),
                             preferred_element_type=jnp.float32)
        tz = tk_ref[0, s, :][None, :]
        valid = smask & (tz != 0)
        m = jnp.max(sc, axis=1, keepdims=True)
        e = jnp.exp(sc - m) * valid.astype(jnp.float32)
        den = jnp.sum(e, axis=1, keepdims=True)
        o = lax.dot_general(e, vv, (((1,), (0,)), ((), ())),
                            preferred_element_type=jnp.float32) / den
        return jnp.concatenate(
            [o[_SB * h:_SB * (h + 1), :] for h in range(_NH)], axis=1)

    at = jnp.concatenate([att(s) for s in range(sub)], axis=0)

    o = dot(_elu(dot(at, dw1[...]) + db1[...]), dw2[...]) + db2[...] + f
    mu = jnp.mean(o, axis=1, keepdims=True)
    d = o - mu
    var = jnp.mean(d * d, axis=1, keepdims=True)
    o_ref[...] = d * lax.rsqrt(var + 1e-5) * lg[...] + lb[...]


def _tc_kwargs(n, k):
    nb = n // _BN
    sub = _BN // _SB
    cm = lambda i: (0, 0)
    in_specs = [
        pl.BlockSpec((_BN, 128), lambda i: (i, 0)),
        pl.BlockSpec((_BN * k, 128), lambda i: (i, 0)),
        pl.BlockSpec((_BN * k, 16), lambda i: (i, 0)),
        pl.BlockSpec((_BN, 16), lambda i: (i, 0)),
        pl.BlockSpec((_BN * k, _BN), cm),
        pl.BlockSpec((1, sub, _SB * k), lambda i: (i, 0, 0)),
        pl.BlockSpec((16, 48), cm), pl.BlockSpec((16, 48), cm),
        pl.BlockSpec((48, 16), cm), pl.BlockSpec((48, 128), cm),
        pl.BlockSpec((1, 16), cm),
        pl.BlockSpec((128, 128), cm), pl.BlockSpec((1, 128), cm),
        pl.BlockSpec((128, 512), cm), pl.BlockSpec((1, 512), cm),
        pl.BlockSpec((16, 128), cm), pl.BlockSpec((1, 128), cm),
        pl.BlockSpec((128, 128), cm), pl.BlockSpec((1, 128), cm),
        pl.BlockSpec((128, 128), cm), pl.BlockSpec((1, 128), cm),
        pl.BlockSpec((128, 128), cm), pl.BlockSpec((1, 128), cm),
        pl.BlockSpec((128, 128), cm), pl.BlockSpec((1, 128), cm),
        pl.BlockSpec((128, 128), cm), pl.BlockSpec((1, 128), cm),
        pl.BlockSpec((512, 128), cm), pl.BlockSpec((1, 128), cm),
        pl.BlockSpec((128, 128), cm), pl.BlockSpec((1, 128), cm),
        pl.BlockSpec((1, 128), cm), pl.BlockSpec((1, 128), cm),
    ]
    return dict(
        grid=(nb,),
        in_specs=in_specs,
        out_specs=pl.BlockSpec((_BN, 128), lambda i: (i, 0)),
        out_shape=jax.ShapeDtypeStruct((n, 128), jnp.float32),
    )


def kernel(features, x, nuv, topk, qf_w1, qf_b1, qf_w2, qf_b2,
           ge_w1, ge_b1, ge_w2, ge_b2, kf_w1, kf_b1, kf_w2, kf_b2,
           vf_w1, vf_b1, vf_w2, vf_b2, df_w1, df_b1, df_w2, df_b2,
           ln_g, ln_b):
    n = features.shape[0]
    k = topk.shape[1]
    f32 = jnp.float32

    geo12 = jnp.concatenate([x.astype(f32), nuv.reshape(n, 9).astype(f32)], axis=1)
    gtab = jnp.concatenate([geo12, jnp.zeros((n, 116), f32)], axis=1)
    ctab = jnp.concatenate([geo12, jnp.zeros((n, 4), f32)], axis=1)
    idx = topk.reshape(-1).astype(jnp.int32)

    sub = _BN // _SB
    tki = topk.astype(jnp.int32).reshape(n // _BN, sub, _SB * k)
    gw1p = jnp.concatenate([ge_w1, jnp.zeros((4, 128), f32)], axis=0)
    p32 = jnp.asarray(
        (np.arange(_BN * k)[:, None] // k == np.arange(_BN)[None, :]).astype(np.float32))
    row = lambda b: b.reshape(1, -1)

    body = functools.partial(_tc_body, sub=sub)
    outs = []
    n0 = 0
    for nc in _CHUNKS:
        gf, gg = _sc_gather(features, gtab, idx[n0 * k:(n0 + nc) * k])
        out_c = pl.pallas_call(body, **_tc_kwargs(nc, k))(
            lax.slice_in_dim(features, n0, n0 + nc),
            gf, gg,
            lax.slice_in_dim(ctab, n0, n0 + nc),
            p32, lax.slice_in_dim(tki, n0 // _BN, (n0 + nc) // _BN),
            jnp.asarray(_PCC), jnp.asarray(_PDC),
            jnp.asarray(_S48), jnp.asarray(_BS48), jnp.asarray(_M012),
            qf_w1, row(qf_b1), qf_w2, row(qf_b2),
            gw1p, row(ge_b1), ge_w2, row(ge_b2),
            kf_w1, row(kf_b1), kf_w2, row(kf_b2),
            vf_w1, row(vf_b1), vf_w2, row(vf_b2),
            df_w1, row(df_b1), df_w2, row(df_b2),
            row(ln_g), row(ln_b))
        outs.append(out_c)
        n0 += nc
    return jnp.concatenate(outs, axis=0)


# speedup vs baseline: 1.1136x; 1.0055x over previous
import functools
import math

import jax
import jax.numpy as jnp
import numpy as np
from jax import lax
from jax.experimental import pallas as pl
from jax.experimental.pallas import tpu as pltpu
from jax.experimental.pallas import tpu_sc as plsc

_NI = 128
_ND = 128
_NH = 4
_KN = 32
_SDK = math.sqrt(float(_ND))
_BN = 80
_SB = 16
_CH = 80
_CHUNKS = (2000, 2000, 2000, 2000, 2000)


def _elu(t):
    return jnp.where(t > 0, t, jnp.exp(jnp.minimum(t, 0.0)) - 1.0)


def _np_consts():
    pcc = np.zeros((16, 48), np.float32)
    pdc = np.zeros((16, 48), np.float32)
    s48 = np.zeros((48, 16), np.float32)
    for b in range(3):
        for j in range(12):
            pcc[3 + 3 * (j % 3) + b, 16 * b + j] = 1.0
            pdc[3 * (j // 3) + b, 16 * b + j] = 1.0
            s48[16 * b + j, j] = 1.0
    bs48 = np.zeros((48, 128), np.float32)
    bs48[0, :] = bs48[16, :] = bs48[32, :] = -0.5
    m012 = np.zeros((1, 16), np.float32)
    m012[0, :3] = 1.0
    return pcc, pdc, s48, bs48, m012


_PCC, _PDC, _S48, _BS48, _M012 = _np_consts()


def _sc_gather(ftab, gtab, idx):
    B = idx.shape[0]
    info = plsc.get_sparse_core_info()
    NC, NS = info.num_cores, info.num_subcores
    NW = NC * NS
    bpw = B // NW
    nch = bpw // _CH
    mesh = plsc.VectorSubcoreMesh(core_axis_name="c", subcore_axis_name="s")

    @functools.partial(
        pl.kernel,
        mesh=mesh,
        out_type=[
            jax.ShapeDtypeStruct((B, 128), jnp.float32),
            jax.ShapeDtypeStruct((B, 16), jnp.float32),
        ],
        scratch_types=[
            pltpu.VMEM((bpw,), jnp.int32),
            pltpu.VMEM((_CH, 128), jnp.float32),
            pltpu.VMEM((_CH, 128), jnp.float32),
            pltpu.VMEM((_CH, 128), jnp.float32),
            pltpu.VMEM((_CH, 128), jnp.float32),
            pltpu.VMEM((_CH, 16), jnp.float32),
            pltpu.SemaphoreType.DMA,
            pltpu.SemaphoreType.DMA,
            pltpu.SemaphoreType.DMA,
            pltpu.SemaphoreType.DMA,
        ],
    )
    def k(ftab_hbm, gtab_hbm, idx_hbm, outf, outg,
          idx_v, fa, ga, fb, gb, gcomp, sfa, sga, sfb, sgb):
        wid = lax.axis_index("s") * NC + lax.axis_index("c")
        base = wid * bpw
        pltpu.sync_copy(idx_hbm.at[pl.ds(base, bpw)], idx_v)

        def fire(c, fbuf, gbuf, sf, sg):
            pltpu.async_copy(ftab_hbm.at[idx_v.at[pl.ds(c * _CH, _CH)]], fbuf, sf)
            pltpu.async_copy(gtab_hbm.at[idx_v.at[pl.ds(c * _CH, _CH)]], gbuf, sg)

        def drain(fbuf, gbuf, sf, sg):
            pltpu.make_async_copy(ftab_hbm.at[pl.ds(0, _CH)], fbuf, sf).wait()
            pltpu.make_async_copy(gtab_hbm.at[pl.ds(0, _CH)], gbuf, sg).wait()

        def process(c, fbuf, gbuf):
            def compact(i, cc):
                gcomp[i, :] = gbuf[i, 0:16]
                return cc

            lax.fori_loop(0, _CH, compact, 0)
            pltpu.sync_copy(fbuf, outf.at[pl.ds(base + c * _CH, _CH)])
            pltpu.sync_copy(gcomp, outg.at[pl.ds(base + c * _CH, _CH)])

        fire(0, fa, ga, sfa, sga)

        def body(t, carry):
            c0 = 2 * t
            fire(c0 + 1, fb, gb, sfb, sgb)
            drain(fa, ga, sfa, sga)
            process(c0, fa, ga)
            fire(c0 + 2, fa, ga, sfa, sga)
            drain(fb, gb, sfb, sgb)
            process(c0 + 1, fb, gb)
            return carry

        lax.fori_loop(0, (nch - 1) // 2, body, 0)
        drain(fa, ga, sfa, sga)
        process(nch - 1, fa, ga)

    return k(ftab, gtab, idx)


def _tc_body(f_ref, gf_ref, gg_ref, cg_ref, p32_ref, tk_ref,
             pcc, pdc, s48, bs48, m012_ref,
             qw1, qb1, qw2, qb2, gw1, gb1, gw2, gb2,
             kw1, kb1, kw2, kb2, vw1, vb1, vw2, vb2,
             dw1, db1, dw2, db2, lg, lb,
             o_ref, *, sub):
    f = f_ref[...]
    dot = functools.partial(jnp.dot, preferred_element_type=jnp.float32)

    t = _elu(dot(f, qw1[...]) + qb1[...])
    qq = (dot(t, qw2[...]) + qb2[...]) * (1.0 / _SDK)

    cg = cg_ref[...]
    e = dot(p32_ref[...], cg)
    gg = gg_ref[...]
    gp = gg - e * m012_ref[...]
    ac = dot(e, pcc[...])
    dc = dot(gp, pdc[...])
    rl = dot(ac * dc, s48[...])
    geo_pre = dot(rl, gw1[...]) + gb1[...]

    dis128 = jnp.exp(dot(dc * dc, bs48[...]))

    gfl = dot(_elu(geo_pre), gw2[...]) + gb2[...]
    gfl = gfl * dis128 * gf_ref[...]

    rows, cols = _NH * _SB, _SB * _KN
    ri = lax.broadcasted_iota(jnp.int32, (rows, cols), 0)
    ci = lax.broadcasted_iota(jnp.int32, (rows, cols), 1)
    smask = (ri % _SB) == (ci // _KN)

    k_all = dot(_elu(dot(gfl, kw1[...]) + kb1[...]), kw2[...]) + kb2[...]
    v_all = dot(_elu(dot(gfl, vw1[...]) + vb1[...]), vw2[...]) + vb2[...]

    def att(s):
        q = lax.slice(qq, (s * _SB, 0), (s * _SB + _SB, 512))
        qs = jnp.concatenate([q[:, 128 * h:128 * (h + 1)] for h in range(_NH)], axis=0)
        kk = lax.slice(k_all, (s * cols, 0), (s * cols + cols, 128))
        vv = lax.slice(v_all, (s * cols, 0), (s * cols + cols, 128))
        sc = lax.dot_general(qs, kk, (((1,), (1,)), ((), ())),
                             preferred_element_type=jnp.float32)
        tz = tk_ref[0, s, :][None, :]
        valid = smask & (tz != 0)
        m = jnp.max(sc, axis=1, keepdims=True)
        e = jnp.exp(sc - m) * valid.astype(jnp.float32)
        den = jnp.sum(e, axis=1, keepdims=True)
        o = lax.dot_general(e, vv, (((1,), (0,)), ((), ())),
                            preferred_element_type=jnp.float32) / den
        return jnp.concatenate(
            [o[_SB * h:_SB * (h + 1), :] for h in range(_NH)], axis=1)

    at = jnp.concatenate([att(s) for s in range(sub)], axis=0)

    o = dot(_elu(dot(at, dw1[...]) + db1[...]), dw2[...]) + db2[...] + f
    mu = jnp.mean(o, axis=1, keepdims=True)
    d = o - mu
    var = jnp.mean(d * d, axis=1, keepdims=True)
    o_ref[...] = d * lax.rsqrt(var + 1e-5) * lg[...] + lb[...]


def _tc_kwargs(n, k):
    nb = n // _BN
    sub = _BN // _SB
    cm = lambda i: (0, 0)
    in_specs = [
        pl.BlockSpec((_BN, 128), lambda i: (i, 0)),
        pl.BlockSpec((_BN * k, 128), lambda i: (i, 0)),
        pl.BlockSpec((_BN * k, 16), lambda i: (i, 0)),
        pl.BlockSpec((_BN, 16), lambda i: (i, 0)),
        pl.BlockSpec((_BN * k, _BN), cm),
        pl.BlockSpec((1, sub, _SB * k), lambda i: (i, 0, 0)),
        pl.BlockSpec((16, 48), cm), pl.BlockSpec((16, 48), cm),
        pl.BlockSpec((48, 16), cm), pl.BlockSpec((48, 128), cm),
        pl.BlockSpec((1, 16), cm),
        pl.BlockSpec((128, 128), cm), pl.BlockSpec((1, 128), cm),
        pl.BlockSpec((128, 512), cm), pl.BlockSpec((1, 512), cm),
        pl.BlockSpec((16, 128), cm), pl.BlockSpec((1, 128), cm),
        pl.BlockSpec((128, 128), cm), pl.BlockSpec((1, 128), cm),
        pl.BlockSpec((128, 128), cm), pl.BlockSpec((1, 128), cm),
        pl.BlockSpec((128, 128), cm), pl.BlockSpec((1, 128), cm),
        pl.BlockSpec((128, 128), cm), pl.BlockSpec((1, 128), cm),
        pl.BlockSpec((128, 128), cm), pl.BlockSpec((1, 128), cm),
        pl.BlockSpec((512, 128), cm), pl.BlockSpec((1, 128), cm),
        pl.BlockSpec((128, 128), cm), pl.BlockSpec((1, 128), cm),
        pl.BlockSpec((1, 128), cm), pl.BlockSpec((1, 128), cm),
    ]
    return dict(
        grid=(nb,),
        in_specs=in_specs,
        out_specs=pl.BlockSpec((_BN, 128), lambda i: (i, 0)),
        out_shape=jax.ShapeDtypeStruct((n, 128), jnp.float32),
    )


def kernel(features, x, nuv, topk, qf_w1, qf_b1, qf_w2, qf_b2,
           ge_w1, ge_b1, ge_w2, ge_b2, kf_w1, kf_b1, kf_w2, kf_b2,
           vf_w1, vf_b1, vf_w2, vf_b2, df_w1, df_b1, df_w2, df_b2,
           ln_g, ln_b):
    n = features.shape[0]
    k = topk.shape[1]
    f32 = jnp.float32

    geo12 = jnp.concatenate([x.astype(f32), nuv.reshape(n, 9).astype(f32)], axis=1)
    gtab = jnp.concatenate([geo12, jnp.zeros((n, 116), f32)], axis=1)
    ctab = jnp.concatenate([geo12, jnp.zeros((n, 4), f32)], axis=1)
    idx = topk.reshape(-1).astype(jnp.int32)

    sub = _BN // _SB
    tki = topk.astype(jnp.int32).reshape(n // _BN, sub, _SB * k)
    gw1p = jnp.concatenate([ge_w1, jnp.zeros((4, 128), f32)], axis=0)
    p32 = jnp.asarray(
        (np.arange(_BN * k)[:, None] // k == np.arange(_BN)[None, :]).astype(np.float32))
    row = lambda b: b.reshape(1, -1)

    body = functools.partial(_tc_body, sub=sub)
    outs = []
    n0 = 0
    for nc in _CHUNKS:
        gf, gg = _sc_gather(features, gtab, idx[n0 * k:(n0 + nc) * k])
        out_c = pl.pallas_call(body, **_tc_kwargs(nc, k))(
            lax.slice_in_dim(features, n0, n0 + nc),
            gf, gg,
            lax.slice_in_dim(ctab, n0, n0 + nc),
            p32, lax.slice_in_dim(tki, n0 // _BN, (n0 + nc) // _BN),
            jnp.asarray(_PCC), jnp.asarray(_PDC),
            jnp.asarray(_S48), jnp.asarray(_BS48), jnp.asarray(_M012),
            qf_w1, row(qf_b1), qf_w2, row(qf_b2),
            gw1p, row(ge_b1), ge_w2, row(ge_b2),
            kf_w1, row(kf_b1), kf_w2, row(kf_b2),
            vf_w1, row(vf_b1), vf_w2, row(vf_b2),
            df_w1, row(df_b1), df_w2, row(df_b2),
            row(ln_g), row(ln_b))
        outs.append(out_c)
        n0 += nc
    return jnp.concatenate(outs, axis=0)
